# baseline (speedup 1.0000x reference)
import jax
import jax.numpy as jnp
from jax.experimental import pallas as pl
from jax.experimental.pallas import tpu as pltpu


def _dma_body(x_ref, o_ref, sem):
    copy = pltpu.make_async_copy(x_ref, o_ref, sem)
    copy.start()
    copy.wait()


def kernel(x, embed_weight, embed_ind, mask, sampled):
    n, d = x.shape
    return pl.pallas_call(
        _dma_body,
        in_specs=[pl.BlockSpec(memory_space=pl.ANY)],
        out_specs=pl.BlockSpec(memory_space=pl.ANY),
        out_shape=jax.ShapeDtypeStruct((n, d), x.dtype),
        scratch_shapes=[pltpu.SemaphoreType.DMA],
    )(x)


# speedup vs baseline: 29.1624x; 29.1624x over previous
import jax
import jax.numpy as jnp
from jax.experimental import pallas as pl

_BLOCK_ROWS = 512


def _copy_body(x_ref, o_ref):
    o_ref[...] = x_ref[...]


def kernel(x, embed_weight, embed_ind, mask, sampled):
    n, d = x.shape
    return pl.pallas_call(
        _copy_body,
        grid=(pl.cdiv(n, _BLOCK_ROWS),),
        in_specs=[pl.BlockSpec((_BLOCK_ROWS, d), lambda i: (i, 0))],
        out_specs=pl.BlockSpec((_BLOCK_ROWS, d), lambda i: (i, 0)),
        out_shape=jax.ShapeDtypeStruct((n, d), x.dtype),
    )(x)


# speedup vs baseline: 44.0783x; 1.5115x over previous
import jax
import jax.numpy as jnp
from jax.experimental import pallas as pl

_BLOCK_ROWS = 2048


def _copy_body(x_ref, o_ref):
    o_ref[...] = x_ref[...]


def kernel(x, embed_weight, embed_ind, mask, sampled):
    n, d = x.shape
    return pl.pallas_call(
        _copy_body,
        grid=(pl.cdiv(n, _BLOCK_ROWS),),
        in_specs=[pl.BlockSpec((_BLOCK_ROWS, d), lambda i: (i, 0))],
        out_specs=pl.BlockSpec((_BLOCK_ROWS, d), lambda i: (i, 0)),
        out_shape=jax.ShapeDtypeStruct((n, d), x.dtype),
    )(x)


# speedup vs baseline: 46.8305x; 1.0624x over previous
import jax
import jax.numpy as jnp
from jax.experimental import pallas as pl

_BLOCK_ROWS = 3112


def _copy_body(x_ref, o_ref):
    o_ref[...] = x_ref[...]


def kernel(x, embed_weight, embed_ind, mask, sampled):
    n, d = x.shape
    return pl.pallas_call(
        _copy_body,
        grid=(pl.cdiv(n, _BLOCK_ROWS),),
        in_specs=[pl.BlockSpec((_BLOCK_ROWS, d), lambda i: (i, 0))],
        out_specs=pl.BlockSpec((_BLOCK_ROWS, d), lambda i: (i, 0)),
        out_shape=jax.ShapeDtypeStruct((n, d), x.dtype),
    )(x)


# speedup vs baseline: 48.6006x; 1.0378x over previous
import jax
import jax.numpy as jnp
from jax.experimental import pallas as pl

_BLOCK_ROWS = 4672


def _copy_body(x_ref, o_ref):
    o_ref[...] = x_ref[...]


def kernel(x, embed_weight, embed_ind, mask, sampled):
    n, d = x.shape
    return pl.pallas_call(
        _copy_body,
        grid=(pl.cdiv(n, _BLOCK_ROWS),),
        in_specs=[pl.BlockSpec((_BLOCK_ROWS, d), lambda i: (i, 0))],
        out_specs=pl.BlockSpec((_BLOCK_ROWS, d), lambda i: (i, 0)),
        out_shape=jax.ShapeDtypeStruct((n, d), x.dtype),
    )(x)
